# Initial kernel scaffold; baseline (speedup 1.0000x reference)
import jax
import jax.numpy as jnp
from jax.experimental import pallas as pl


def kernel(h, e, edge_index, W_emb, b_emb, Wl, bl, gamma, beta):
    raise NotImplementedError("write your pallas kernel here")


# speedup vs baseline: 3.6242x; 3.6242x over previous
import jax
import jax.numpy as jnp
from jax import lax
from jax.experimental import pallas as pl
from jax.experimental.pallas import tpu as pltpu
from jax.experimental.pallas import tpu_sc as plsc

N = 10000
D = 128
H = D // 2
E = 320000
L = 4

NC = 2
NS = 16

B = 128
NB = 8
NG = 4
EPW = 20480
BPW = EPW // B
NMACRO = BPW // NB
EPAD = EPW * NS
NP = 10240
ZR = NP // NS


def _prop_body(srcm, dstm, xs_hbm, zeros_hbm, out_hbm,
               src_v, dst_v, rows_v, acc, gsem, ssem):
    c = lax.axis_index("c")
    s = lax.axis_index("s")
    pltpu.sync_copy(zeros_hbm, acc.at[pl.ds(s * ZR, ZR)])
    plsc.subcore_barrier()

    def step(i, carry):
        row0 = s * BPW + i * NB
        pltpu.sync_copy(srcm.at[pl.ds(row0, NB)], src_v)
        pltpu.sync_copy(dstm.at[pl.ds(row0, NB)], dst_v)
        for g in range(NB // NG):
            gathers = [
                pltpu.async_copy(xs_hbm.at[c].at[src_v.at[g * NG + j]],
                                 rows_v.at[pl.ds(j * B, B)], gsem)
                for j in range(NG)
            ]
            for cp in gathers:
                cp.wait()
            scatters = [
                pltpu.async_copy(rows_v.at[pl.ds(j * B, B)],
                                 acc.at[dst_v.at[g * NG + j]], ssem, add=True)
                for j in range(NG)
            ]
            for cp in scatters:
                cp.wait()
        return carry

    lax.fori_loop(0, NMACRO, step, 0)
    plsc.subcore_barrier()
    pltpu.sync_copy(acc.at[pl.ds(s * ZR, ZR)],
                    out_hbm.at[c, pl.ds(s * ZR, ZR)])


_prop = pl.kernel(
    _prop_body,
    out_type=jax.ShapeDtypeStruct((NC, NP, H), jnp.float32),
    mesh=plsc.VectorSubcoreMesh(core_axis_name="c", subcore_axis_name="s",
                                num_cores=NC, num_subcores=NS),
    scratch_types=[
        pltpu.VMEM((NB, B), jnp.int32),
        pltpu.VMEM((NB, B), jnp.int32),
        pltpu.VMEM((NG * B, H), jnp.float32),
        pltpu.VMEM_SHARED((NP, H), jnp.float32),
        pltpu.SemaphoreType.DMA,
        pltpu.SemaphoreType.DMA,
    ],
    compiler_params=pltpu.CompilerParams(use_tc_tiling_on_sc=False),
)


def _embed_body(h_ref, w_ref, b_ref, o_ref):
    x = (jnp.dot(h_ref[...], w_ref[...],
                 preferred_element_type=jnp.float32) + b_ref[...])
    o_ref[0] = x[:, :H]
    o_ref[1] = x[:, H:]


_embed = pl.pallas_call(
    _embed_body,
    out_shape=jax.ShapeDtypeStruct((NC, N, H), jnp.float32),
)


def _tail_body(xs_ref, y1_ref, y2_ref, w_ref, b_ref, g_ref, bt_ref,
               o_ref, of_ref):
    t = (jnp.dot(xs_ref[0], w_ref[0, :H], preferred_element_type=jnp.float32)
         + jnp.dot(xs_ref[1], w_ref[0, H:], preferred_element_type=jnp.float32)
         + jnp.dot(y1_ref[0, :N], w_ref[1, :H],
                   preferred_element_type=jnp.float32)
         + jnp.dot(y1_ref[1, :N], w_ref[1, H:],
                   preferred_element_type=jnp.float32)
         + jnp.dot(y2_ref[0, :N], w_ref[2, :H],
                   preferred_element_type=jnp.float32)
         + jnp.dot(y2_ref[1, :N], w_ref[2, H:],
                   preferred_element_type=jnp.float32)
         + b_ref[...])
    mu = jnp.mean(t, axis=0, keepdims=True)
    var = jnp.mean((t - mu) * (t - mu), axis=0, keepdims=True)
    t = (t - mu) * lax.rsqrt(var + 1e-5) * g_ref[...] + bt_ref[...]
    t = jnp.maximum(t, 0.0)
    ra = t[:, :H] + xs_ref[0]
    rb = t[:, H:] + xs_ref[1]
    o_ref[0] = ra
    o_ref[1] = rb
    of_ref[...] = jnp.concatenate([ra, rb], axis=1)


_tail = pl.pallas_call(
    _tail_body,
    out_shape=(jax.ShapeDtypeStruct((NC, N, H), jnp.float32),
               jax.ShapeDtypeStruct((N, D), jnp.float32)),
    compiler_params=pltpu.CompilerParams(vmem_limit_bytes=100 * 1024 * 1024),
)


def kernel(h, e, edge_index, W_emb, b_emb, Wl, bl, gamma, beta):
    src = edge_index[0]
    dst = edge_index[1]
    pad = EPAD - E
    src_p = jnp.concatenate([src, jnp.zeros((pad,), jnp.int32)])
    dst_p = jnp.concatenate([dst, jnp.full((pad,), N, jnp.int32)])
    srcm = src_p.reshape(-1, B)
    dstm = dst_p.reshape(-1, B)
    zeros = jnp.zeros((ZR, H), jnp.float32)

    xs = _embed(h, W_emb, b_emb.reshape(1, D))
    xf = None
    for l in range(L):
        y1 = _prop(srcm, dstm, xs, zeros)
        y2 = _prop(srcm, dstm, y1, zeros)
        xs, xf = _tail(xs, y1, y2, Wl[l],
                       (bl[l, 0] + bl[l, 1] + bl[l, 2]).reshape(1, D),
                       gamma[l].reshape(1, D), beta[l].reshape(1, D))
    return xf


# speedup vs baseline: 3.8203x; 1.0541x over previous
import jax
import jax.numpy as jnp
from jax import lax
from jax.experimental import pallas as pl
from jax.experimental.pallas import tpu as pltpu
from jax.experimental.pallas import tpu_sc as plsc

N = 10000
D = 128
H = D // 2
E = 320000
L = 4

NC = 2
NS = 16

B = 128
NB = 8
NG = 4
EPW = 20480
BPW = EPW // B
NMACRO = BPW // NB
EPAD = EPW * NS
NP = 10240
ZR = NP // NS


NPAIR = BPW // NB
PR = 2 * NB


def _prop_body(sdm, xs_hbm, zeros_hbm, out_hbm,
               i0, i1, ra, rb, acc, isem, gsa, gsb, ssa, ssb):
    c = lax.axis_index("c")
    s = lax.axis_index("s")
    pltpu.sync_copy(zeros_hbm, acc.at[pl.ds(s * ZR, ZR)])
    plsc.subcore_barrier()

    base = s * (BPW * 2)

    def do_pair(prow, ibuf):
        ga = [pltpu.async_copy(xs_hbm.at[c].at[ibuf.at[2 * j]],
                               ra.at[pl.ds(j * B, B)], gsa)
              for j in range(NG)]
        gb = [pltpu.async_copy(xs_hbm.at[c].at[ibuf.at[2 * (NG + j)]],
                               rb.at[pl.ds(j * B, B)], gsb)
              for j in range(NG)]
        for cp in ga:
            cp.wait()
        sa = [pltpu.async_copy(ra.at[pl.ds(j * B, B)],
                               acc.at[ibuf.at[2 * j + 1]], ssa, add=True)
              for j in range(NG)]
        for cp in gb:
            cp.wait()
        sb = [pltpu.async_copy(rb.at[pl.ds(j * B, B)],
                               acc.at[ibuf.at[2 * (NG + j) + 1]], ssb,
                               add=True)
              for j in range(NG)]
        for cp in sa:
            cp.wait()
        for cp in sb:
            cp.wait()

    pltpu.sync_copy(sdm.at[pl.ds(base, PR)], i0)

    def step(i, carry):
        p = 2 * i
        pf1 = pltpu.async_copy(sdm.at[pl.ds(base + (p + 1) * PR, PR)],
                               i1, isem)
        do_pair(p, i0)
        pf1.wait()
        @pl.when(i < NPAIR // 2 - 1)
        def _():
            pltpu.async_copy(sdm.at[pl.ds(base + (p + 2) * PR, PR)],
                             i0, isem).wait()
        do_pair(p + 1, i1)
        return carry

    lax.fori_loop(0, NPAIR // 2, step, 0)
    plsc.subcore_barrier()
    pltpu.sync_copy(acc.at[pl.ds(s * ZR, ZR)],
                    out_hbm.at[c, pl.ds(s * ZR, ZR)])


_prop = pl.kernel(
    _prop_body,
    out_type=jax.ShapeDtypeStruct((NC, NP, H), jnp.float32),
    mesh=plsc.VectorSubcoreMesh(core_axis_name="c", subcore_axis_name="s",
                                num_cores=NC, num_subcores=NS),
    scratch_types=[
        pltpu.VMEM((PR, B), jnp.int32),
        pltpu.VMEM((PR, B), jnp.int32),
        pltpu.VMEM((NG * B, H), jnp.float32),
        pltpu.VMEM((NG * B, H), jnp.float32),
        pltpu.VMEM_SHARED((NP, H), jnp.float32),
        pltpu.SemaphoreType.DMA,
        pltpu.SemaphoreType.DMA,
        pltpu.SemaphoreType.DMA,
        pltpu.SemaphoreType.DMA,
        pltpu.SemaphoreType.DMA,
    ],
    compiler_params=pltpu.CompilerParams(use_tc_tiling_on_sc=False),
)


def _embed_body(h_ref, w_ref, b_ref, o_ref):
    x = (jnp.dot(h_ref[...], w_ref[...],
                 preferred_element_type=jnp.float32) + b_ref[...])
    o_ref[0] = x[:, :H]
    o_ref[1] = x[:, H:]


_embed = pl.pallas_call(
    _embed_body,
    out_shape=jax.ShapeDtypeStruct((NC, N, H), jnp.float32),
)


def _tail_body(xs_ref, y1_ref, y2_ref, w_ref, b_ref, g_ref, bt_ref,
               o_ref, of_ref):
    t = (jnp.dot(xs_ref[0], w_ref[0, :H], preferred_element_type=jnp.float32)
         + jnp.dot(xs_ref[1], w_ref[0, H:], preferred_element_type=jnp.float32)
         + jnp.dot(y1_ref[0, :N], w_ref[1, :H],
                   preferred_element_type=jnp.float32)
         + jnp.dot(y1_ref[1, :N], w_ref[1, H:],
                   preferred_element_type=jnp.float32)
         + jnp.dot(y2_ref[0, :N], w_ref[2, :H],
                   preferred_element_type=jnp.float32)
         + jnp.dot(y2_ref[1, :N], w_ref[2, H:],
                   preferred_element_type=jnp.float32)
         + b_ref[...])
    mu = jnp.mean(t, axis=0, keepdims=True)
    var = jnp.mean((t - mu) * (t - mu), axis=0, keepdims=True)
    t = (t - mu) * lax.rsqrt(var + 1e-5) * g_ref[...] + bt_ref[...]
    t = jnp.maximum(t, 0.0)
    ra = t[:, :H] + xs_ref[0]
    rb = t[:, H:] + xs_ref[1]
    o_ref[0] = ra
    o_ref[1] = rb
    of_ref[...] = jnp.concatenate([ra, rb], axis=1)


_tail = pl.pallas_call(
    _tail_body,
    out_shape=(jax.ShapeDtypeStruct((NC, N, H), jnp.float32),
               jax.ShapeDtypeStruct((N, D), jnp.float32)),
    compiler_params=pltpu.CompilerParams(vmem_limit_bytes=100 * 1024 * 1024),
)


def kernel(h, e, edge_index, W_emb, b_emb, Wl, bl, gamma, beta):
    src = edge_index[0]
    dst = edge_index[1]
    pad = EPAD - E
    src_p = jnp.concatenate([src, jnp.zeros((pad,), jnp.int32)])
    dst_p = jnp.concatenate([dst, jnp.full((pad,), N, jnp.int32)])
    srcm = src_p.reshape(-1, B)
    dstm = dst_p.reshape(-1, B)
    sdm = jnp.stack([srcm, dstm], axis=1).reshape(-1, B)
    zeros = jnp.zeros((ZR, H), jnp.float32)

    xs = _embed(h, W_emb, b_emb.reshape(1, D))
    xf = None
    for l in range(L):
        y1 = _prop(sdm, xs, zeros)
        y2 = _prop(sdm, y1, zeros)
        xs, xf = _tail(xs, y1, y2, Wl[l],
                       (bl[l, 0] + bl[l, 1] + bl[l, 2]).reshape(1, D),
                       gamma[l].reshape(1, D), beta[l].reshape(1, D))
    return xf
